# baseline (device time: 8398 ns/iter reference)
import jax
import jax.numpy as jnp
from jax import lax
from jax.experimental import pallas as pl
from jax.experimental.pallas import tpu as pltpu

N_CHUNKS = 2


def kernel(x):
    m, n = x.shape
    mc = m // N_CHUNKS

    def body(x_hbm, out_hbm, x_vmem, comm_ref, res_ref,
             in_sems, out_sems, send_sems, recv_sems):
        my_x = lax.axis_index("x")
        my_y = lax.axis_index("y")
        nbr = (my_x, 1 - my_y)

        barrier_sem = pltpu.get_barrier_semaphore()
        pl.semaphore_signal(
            barrier_sem, inc=1, device_id=nbr,
            device_id_type=pl.DeviceIdType.MESH,
        )

        in_copies = []
        for c in range(N_CHUNKS):
            rows = pl.ds(c * mc, mc)
            cp = pltpu.make_async_copy(x_hbm.at[rows], x_vmem.at[rows], in_sems.at[c])
            cp.start()
            in_copies.append(cp)

        rdmas = []
        for c in range(N_CHUNKS):
            rows = pl.ds(c * mc, mc)
            in_copies[c].wait()
            comm_ref[0, rows, :] = jnp.max(x_vmem[rows, :], axis=1, keepdims=True)
            if c == 0:
                pl.semaphore_wait(barrier_sem, 1)
            rdma = pltpu.make_async_remote_copy(
                src_ref=comm_ref.at[0, rows],
                dst_ref=comm_ref.at[1, rows],
                send_sem=send_sems.at[c],
                recv_sem=recv_sems.at[c],
                device_id=nbr,
                device_id_type=pl.DeviceIdType.MESH,
            )
            rdma.start()
            rdmas.append(rdma)

        out_copies = []
        for c in range(N_CHUNKS):
            rows = pl.ds(c * mc, mc)
            rdmas[c].wait_recv()
            res_ref[rows, :] = jnp.maximum(comm_ref[0, rows, :], comm_ref[1, rows, :])
            cp = pltpu.make_async_copy(res_ref.at[rows], out_hbm.at[rows], out_sems.at[c])
            cp.start()
            out_copies.append(cp)

        for c in range(N_CHUNKS):
            out_copies[c].wait()
            rdmas[c].wait_send()

    return pl.pallas_call(
        body,
        out_shape=jax.ShapeDtypeStruct((m, 1), x.dtype),
        in_specs=[pl.BlockSpec(memory_space=pltpu.MemorySpace.HBM)],
        out_specs=pl.BlockSpec(memory_space=pltpu.MemorySpace.HBM),
        scratch_shapes=[
            pltpu.VMEM((m, n), x.dtype),
            pltpu.VMEM((2, m, 1), x.dtype),
            pltpu.VMEM((m, 1), x.dtype),
            pltpu.SemaphoreType.DMA((N_CHUNKS,)),
            pltpu.SemaphoreType.DMA((N_CHUNKS,)),
            pltpu.SemaphoreType.DMA((N_CHUNKS,)),
            pltpu.SemaphoreType.DMA((N_CHUNKS,)),
        ],
        compiler_params=pltpu.CompilerParams(collective_id=0),
    )(pltpu.with_memory_space_constraint(x, pltpu.MemorySpace.HBM))


# device time: 5409 ns/iter; 1.5526x vs baseline; 1.5526x over previous
import jax
import jax.numpy as jnp
from jax import lax
from jax.experimental import pallas as pl
from jax.experimental.pallas import tpu as pltpu


def kernel(x):
    m, n = x.shape
    rows, lanes = m // 128, 128

    def body(x_hbm, out_hbm, x_vmem, comm_ref, res_ref,
             in_sem, out_sem, send_sem, recv_sem):
        my_x = lax.axis_index("x")
        my_y = lax.axis_index("y")
        nbr = (my_x, 1 - my_y)

        barrier_sem = pltpu.get_barrier_semaphore()
        pl.semaphore_signal(
            barrier_sem, inc=1, device_id=nbr,
            device_id_type=pl.DeviceIdType.MESH,
        )

        in_copy = pltpu.make_async_copy(x_hbm, x_vmem, in_sem)
        in_copy.start()
        in_copy.wait()
        comm_ref[0, :, :] = jnp.max(x_vmem[:, :], axis=1).reshape(rows, lanes)

        pl.semaphore_wait(barrier_sem, 1)

        rdma = pltpu.make_async_remote_copy(
            src_ref=comm_ref.at[0],
            dst_ref=comm_ref.at[1],
            send_sem=send_sem,
            recv_sem=recv_sem,
            device_id=nbr,
            device_id_type=pl.DeviceIdType.MESH,
        )
        rdma.start()
        rdma.wait_recv()

        res_ref[:, :] = jnp.maximum(comm_ref[0, :, :], comm_ref[1, :, :])
        out_copy = pltpu.make_async_copy(res_ref, out_hbm, out_sem)
        out_copy.start()
        out_copy.wait()
        rdma.wait_send()

    dense = pl.pallas_call(
        body,
        out_shape=jax.ShapeDtypeStruct((rows, lanes), x.dtype),
        in_specs=[pl.BlockSpec(memory_space=pltpu.MemorySpace.HBM)],
        out_specs=pl.BlockSpec(memory_space=pltpu.MemorySpace.HBM),
        scratch_shapes=[
            pltpu.VMEM((m, n), x.dtype),
            pltpu.VMEM((2, rows, lanes), x.dtype),
            pltpu.VMEM((rows, lanes), x.dtype),
            pltpu.SemaphoreType.DMA,
            pltpu.SemaphoreType.DMA,
            pltpu.SemaphoreType.DMA,
            pltpu.SemaphoreType.DMA,
        ],
        compiler_params=pltpu.CompilerParams(collective_id=0),
    )(pltpu.with_memory_space_constraint(x, pltpu.MemorySpace.HBM))
    return dense.reshape(m, 1)
